# Initial kernel scaffold; baseline (speedup 1.0000x reference)
import jax
import jax.numpy as jnp
from jax.experimental import pallas as pl


def kernel(x, edge_index, edge_weight, W_neigh, W_self, b_self):
    raise NotImplementedError("write your pallas kernel here")


# speedup vs baseline: 1.0059x; 1.0059x over previous
import jax
import jax.numpy as jnp
from jax import lax
from jax.experimental import pallas as pl

N_NODES = 10000
D = 256


def _mm_body(x_ref, a_ref, wst_ref, wnt_ref, b_ref, o_ref):
    o_ref[...] = (
        jnp.dot(x_ref[...], wst_ref[...],
                preferred_element_type=jnp.float32,
                precision=lax.Precision.HIGHEST)
        + jnp.dot(a_ref[...], wnt_ref[...],
                  preferred_element_type=jnp.float32,
                  precision=lax.Precision.HIGHEST)
        + b_ref[...]
    )


def _tc_out(x, agg, W_neigh, W_self, b_self):
    blk = 1000
    return pl.pallas_call(
        _mm_body,
        grid=(N_NODES // blk,),
        in_specs=[
            pl.BlockSpec((blk, D), lambda i: (i, 0)),
            pl.BlockSpec((blk, D), lambda i: (i, 0)),
            pl.BlockSpec((D, D), lambda i: (0, 0)),
            pl.BlockSpec((D, D), lambda i: (0, 0)),
            pl.BlockSpec((1, D), lambda i: (0, 0)),
        ],
        out_specs=pl.BlockSpec((blk, D), lambda i: (i, 0)),
        out_shape=jax.ShapeDtypeStruct((N_NODES, D), jnp.float32),
    )(x, agg, W_self.T, W_neigh.T, b_self[None, :])


def kernel(x, edge_index, edge_weight, W_neigh, W_self, b_self):
    src = edge_index[0].astype(jnp.int32)
    dst = edge_index[1].astype(jnp.int32)
    msg = jnp.take(x, src, axis=0) * edge_weight[:, None]
    agg = jnp.zeros((N_NODES, D), jnp.float32).at[dst].add(msg)
    return _tc_out(x, agg, W_neigh, W_self, b_self)


# speedup vs baseline: 2.3580x; 2.3443x over previous
import functools

import jax
import jax.numpy as jnp
from jax import lax
from jax.experimental import pallas as pl
from jax.experimental.pallas import tpu as pltpu
from jax.experimental.pallas import tpu_sc as plsc

N_NODES = 10000
D = 256
DH = 128
N_EDGES = 160000

NC = 2
NS = 16

C = 128
EPS = 10240
E_PAD = NS * EPS
CHUNKS = EPS // C

ZR = 104
ZN = 6


@functools.lru_cache(maxsize=None)
def _build_sc_agg():
    mesh = plsc.VectorSubcoreMesh(
        core_axis_name="c", subcore_axis_name="s",
        num_cores=NC, num_subcores=NS)
    return functools.partial(
        pl.kernel,
        mesh=mesh,
        out_type=jax.ShapeDtypeStruct((NC * N_NODES, DH), jnp.float32),
        scratch_types=[
            pltpu.VMEM((C,), jnp.int32),
            pltpu.VMEM((C,), jnp.int32),
            pltpu.VMEM((C,), jnp.float32),
            pltpu.VMEM((C, DH), jnp.float32),
            pltpu.VMEM((ZR, DH), jnp.float32),
            pltpu.VMEM_SHARED((N_NODES, DH), jnp.float32),
            pltpu.SemaphoreType.DMA,
        ],
    )(_sc_agg_body)


def _sc_agg_body(xh_hbm, src_hbm, dst_hbm, w_hbm, out_hbm,
                 src_v, dst_v, w_v, rows_v, zero_v, acc_s, sem):
    cid = lax.axis_index("c")
    sid = lax.axis_index("s")

    def zrow(i, carry):
        for j in range(DH // 16):
            zero_v[i, pl.ds(j * 16, 16)] = jnp.zeros((16,), jnp.float32)
        return carry
    lax.fori_loop(0, ZR, zrow, 0)

    zbase = sid * 624

    def zcopy(r, carry):
        pltpu.sync_copy(zero_v, acc_s.at[pl.ds(zbase + r * ZR, ZR)])
        return carry
    lax.fori_loop(0, ZN, zcopy, 0)

    @pl.when(sid == NS - 1)
    def _tail():
        pltpu.sync_copy(zero_v.at[pl.ds(0, 16)],
                        acc_s.at[pl.ds(zbase + ZN * ZR, 16)])

    plsc.subcore_barrier()

    base = sid * EPS
    row_off = jnp.full((16,), cid * N_NODES, jnp.int32)

    def chunk(g, carry):
        off = base + g * C
        pltpu.sync_copy(src_hbm.at[pl.ds(off, C)], src_v)
        pltpu.sync_copy(dst_hbm.at[pl.ds(off, C)], dst_v)
        pltpu.sync_copy(w_hbm.at[pl.ds(off, C)], w_v)

        for q in range(C // 16):
            sl = pl.ds(q * 16, 16)
            src_v[sl] = src_v[sl] + row_off

        pltpu.async_copy(xh_hbm.at[src_v], rows_v, sem).wait()

        def edge_grp(g2, c2):
            wv = w_v[pl.ds(g2 * 16, 16)]
            for k2 in range(16):
                e = g2 * 16 + k2
                wb = jnp.broadcast_to(wv[k2], (16,))
                for j in range(DH // 16):
                    sl = pl.ds(j * 16, 16)
                    rows_v[e, sl] = rows_v[e, sl] * wb
            return c2
        lax.fori_loop(0, C // 16, edge_grp, 0)

        pltpu.sync_copy(rows_v, acc_s.at[dst_v], add=True)
        return carry
    lax.fori_loop(0, CHUNKS, chunk, 0)

    plsc.subcore_barrier()

    obase = cid * N_NODES + zbase

    def ocopy(r, carry):
        pltpu.sync_copy(acc_s.at[pl.ds(zbase + r * ZR, ZR)],
                        out_hbm.at[pl.ds(obase + r * ZR, ZR)])
        return carry
    lax.fori_loop(0, ZN, ocopy, 0)

    @pl.when(sid == NS - 1)
    def _otail():
        pltpu.sync_copy(acc_s.at[pl.ds(zbase + ZN * ZR, 16)],
                        out_hbm.at[pl.ds(obase + ZN * ZR, 16)])


def _mm_body(x_ref, a0_ref, a1_ref, wst_ref, wn0_ref, wn1_ref, b_ref, o_ref):
    o_ref[...] = (
        jnp.dot(x_ref[...], wst_ref[...],
                preferred_element_type=jnp.float32,
                precision=lax.Precision.HIGHEST)
        + jnp.dot(a0_ref[...], wn0_ref[...],
                  preferred_element_type=jnp.float32,
                  precision=lax.Precision.HIGHEST)
        + jnp.dot(a1_ref[...], wn1_ref[...],
                  preferred_element_type=jnp.float32,
                  precision=lax.Precision.HIGHEST)
        + b_ref[...]
    )


def _tc_out(x, agg, W_neigh, W_self, b_self):
    blk = 1000
    nb = N_NODES // blk
    wnt = W_neigh.T
    return pl.pallas_call(
        _mm_body,
        grid=(nb,),
        in_specs=[
            pl.BlockSpec((blk, D), lambda i: (i, 0)),
            pl.BlockSpec((blk, DH), lambda i: (i, 0)),
            pl.BlockSpec((blk, DH), lambda i, _nb=nb: (i + _nb, 0)),
            pl.BlockSpec((D, D), lambda i: (0, 0)),
            pl.BlockSpec((DH, D), lambda i: (0, 0)),
            pl.BlockSpec((DH, D), lambda i: (0, 0)),
            pl.BlockSpec((1, D), lambda i: (0, 0)),
        ],
        out_specs=pl.BlockSpec((blk, D), lambda i: (i, 0)),
        out_shape=jax.ShapeDtypeStruct((N_NODES, D), jnp.float32),
    )(x, agg, agg, W_self.T, wnt[:DH], wnt[DH:], b_self[None, :])


def kernel(x, edge_index, edge_weight, W_neigh, W_self, b_self):
    xh = jnp.concatenate([x[:, :DH], x[:, DH:]], axis=0)
    src = jnp.zeros((E_PAD,), jnp.int32).at[:N_EDGES].set(
        edge_index[0].astype(jnp.int32))
    dst = jnp.zeros((E_PAD,), jnp.int32).at[:N_EDGES].set(
        edge_index[1].astype(jnp.int32))
    w = jnp.zeros((E_PAD,), jnp.float32).at[:N_EDGES].set(edge_weight)
    agg = _build_sc_agg()(xh, src, dst, w)
    return _tc_out(x, agg, W_neigh, W_self, b_self)


# speedup vs baseline: 3.3674x; 1.4281x over previous
import functools

import jax
import jax.numpy as jnp
from jax import lax
from jax.experimental import pallas as pl
from jax.experimental.pallas import tpu as pltpu
from jax.experimental.pallas import tpu_sc as plsc

N_NODES = 10000
D = 256
DH = 128
N_EDGES = 160000

NC = 2
NS = 16

C = 128
EPS = 10240
E_PAD = NS * EPS
CHUNKS = EPS // C
HC = CHUNKS // 2
HIT = HC // 2


@functools.lru_cache(maxsize=None)
def _build_sc_agg():
    mesh = plsc.VectorSubcoreMesh(
        core_axis_name="c", subcore_axis_name="s",
        num_cores=NC, num_subcores=NS)
    return functools.partial(
        pl.kernel,
        mesh=mesh,
        out_type=jax.ShapeDtypeStruct((NC * N_NODES, DH), jnp.float32),
        scratch_types=[
            pltpu.VMEM((HC, C), jnp.int32),
            pltpu.VMEM((HC, C), jnp.int32),
            pltpu.VMEM((HC, C), jnp.float32),
            pltpu.VMEM((C, DH), jnp.float32),
            pltpu.VMEM((C, DH), jnp.float32),
            pltpu.VMEM_SHARED((N_NODES, DH), jnp.float32),
            pltpu.SemaphoreType.DMA,
            pltpu.SemaphoreType.DMA,
        ],
    )(_sc_agg_body)


def _sc_agg_body(xh_hbm, src_hbm, dst_hbm, w_hbm, out_hbm,
                 src_a, dst_a, w_a, rows_a, rows_b, acc_s,
                 sem_a, sem_b):
    cid = lax.axis_index("c")
    sid = lax.axis_index("s")

    def zrow(i, carry):
        for j in range(DH // 16):
            rows_a[i, pl.ds(j * 16, 16)] = jnp.zeros((16,), jnp.float32)
        return carry
    lax.fori_loop(0, C, zrow, 0)

    zbase = sid * 624

    def zcopy(r, carry):
        pltpu.sync_copy(rows_a, acc_s.at[pl.ds(zbase + r * C, C)])
        return carry
    lax.fori_loop(0, 4, zcopy, 0)
    pltpu.sync_copy(rows_a.at[pl.ds(0, 112)],
                    acc_s.at[pl.ds(zbase + 4 * C, 112)])

    @pl.when(sid == NS - 1)
    def _tail():
        pltpu.sync_copy(rows_a.at[pl.ds(0, 16)],
                        acc_s.at[pl.ds(zbase + 624, 16)])

    plsc.subcore_barrier()

    rbase = sid * CHUNKS

    def process(buf, g):
        def edge_grp(g2, c2):
            wv = w_a[g, pl.ds(g2 * 16, 16)]
            for k2 in range(16):
                e = g2 * 16 + k2
                wb = jnp.broadcast_to(wv[k2], (16,))
                for j in range(DH // 16):
                    sl = pl.ds(j * 16, 16)
                    buf[e, sl] = buf[e, sl] * wb
            return c2
        lax.fori_loop(0, C // 16, edge_grp, 0)
        pltpu.sync_copy(buf, acc_s.at[dst_a.at[g]], add=True)

    for h in range(CHUNKS // HC):
        hbase = rbase + h * HC
        pltpu.sync_copy(src_hbm.at[pl.ds(cid * (NS * CHUNKS) + hbase, HC)],
                        src_a)
        pltpu.sync_copy(dst_hbm.at[pl.ds(hbase, HC)], dst_a)
        pltpu.sync_copy(w_hbm.at[pl.ds(hbase, HC)], w_a)

        pltpu.async_copy(xh_hbm.at[src_a.at[0]], rows_a, sem_a)

        def pipe(t, carry):
            g0 = 2 * t
            pltpu.async_copy(xh_hbm.at[src_a.at[g0 + 1]], rows_b, sem_b)
            pltpu.make_async_copy(xh_hbm.at[src_a.at[0]], rows_a, sem_a).wait()
            process(rows_a, g0)

            @pl.when(t + 1 < HIT)
            def _next():
                pltpu.async_copy(xh_hbm.at[src_a.at[g0 + 2]], rows_a, sem_a)

            pltpu.make_async_copy(xh_hbm.at[src_a.at[0]], rows_b, sem_b).wait()
            process(rows_b, g0 + 1)
            return carry
        lax.fori_loop(0, HIT, pipe, 0)

    plsc.subcore_barrier()

    obase = cid * N_NODES + zbase

    def ocopy(r, carry):
        pltpu.sync_copy(acc_s.at[pl.ds(zbase + r * C, C)],
                        out_hbm.at[pl.ds(obase + r * C, C)])
        return carry
    lax.fori_loop(0, 4, ocopy, 0)
    pltpu.sync_copy(acc_s.at[pl.ds(zbase + 4 * C, 112)],
                    out_hbm.at[pl.ds(obase + 4 * C, 112)])

    @pl.when(sid == NS - 1)
    def _otail():
        pltpu.sync_copy(acc_s.at[pl.ds(zbase + 624, 16)],
                        out_hbm.at[pl.ds(obase + 624, 16)])


def _mm_body(x_ref, a0_ref, a1_ref, wst_ref, wn0_ref, wn1_ref, b_ref, o_ref):
    o_ref[...] = (
        jnp.dot(x_ref[...], wst_ref[...],
                preferred_element_type=jnp.float32,
                precision=lax.Precision.HIGHEST)
        + jnp.dot(a0_ref[...], wn0_ref[...],
                  preferred_element_type=jnp.float32,
                  precision=lax.Precision.HIGHEST)
        + jnp.dot(a1_ref[...], wn1_ref[...],
                  preferred_element_type=jnp.float32,
                  precision=lax.Precision.HIGHEST)
        + b_ref[...]
    )


def _tc_out(x, agg, W_neigh, W_self, b_self):
    blk = 1000
    nb = N_NODES // blk
    wnt = W_neigh.T
    return pl.pallas_call(
        _mm_body,
        grid=(nb,),
        in_specs=[
            pl.BlockSpec((blk, D), lambda i: (i, 0)),
            pl.BlockSpec((blk, DH), lambda i: (i, 0)),
            pl.BlockSpec((blk, DH), lambda i, _nb=nb: (i + _nb, 0)),
            pl.BlockSpec((D, D), lambda i: (0, 0)),
            pl.BlockSpec((DH, D), lambda i: (0, 0)),
            pl.BlockSpec((DH, D), lambda i: (0, 0)),
            pl.BlockSpec((1, D), lambda i: (0, 0)),
        ],
        out_specs=pl.BlockSpec((blk, D), lambda i: (i, 0)),
        out_shape=jax.ShapeDtypeStruct((N_NODES, D), jnp.float32),
    )(x, agg, agg, W_self.T, wnt[:DH], wnt[DH:], b_self[None, :])


def kernel(x, edge_index, edge_weight, W_neigh, W_self, b_self):
    xh = jnp.concatenate([x[:, :DH], x[:, DH:]], axis=0)
    src = jnp.zeros((E_PAD,), jnp.int32).at[:N_EDGES].set(
        edge_index[0].astype(jnp.int32))
    src2 = jnp.stack([src, src + N_NODES]).reshape(2 * NS * CHUNKS, C)
    dst = jnp.zeros((E_PAD,), jnp.int32).at[:N_EDGES].set(
        edge_index[1].astype(jnp.int32)).reshape(NS * CHUNKS, C)
    w = jnp.zeros((E_PAD,), jnp.float32).at[:N_EDGES].set(
        edge_weight).reshape(NS * CHUNKS, C)
    agg = _build_sc_agg()(xh, src2, dst, w)
    return _tc_out(x, agg, W_neigh, W_self, b_self)
